# baseline (speedup 1.0000x reference)
import functools

import jax
import jax.numpy as jnp
from jax import lax
from jax.experimental import pallas as pl
from jax.experimental.pallas import tpu as pltpu
from jax.experimental.pallas import tpu_sc as plsc


def kernel(player_ids, opening_ids, player_emb, opening_emb, opening_bias):
    B = player_ids.shape[0]
    D = player_emb.shape[1]
    O = opening_emb.shape[0]

    info = plsc.get_sparse_core_info()
    NC, NS, L = info.num_cores, info.num_subcores, info.num_lanes
    NW = NC * NS
    b_per_w = B // NW
    C = 128
    n_sub = b_per_w // C
    n_grp = C // L

    mesh = plsc.VectorSubcoreMesh(core_axis_name="c", subcore_axis_name="s")

    @functools.partial(
        pl.kernel,
        mesh=mesh,
        compiler_params=pltpu.CompilerParams(needs_layout_passes=False),
        out_type=jax.ShapeDtypeStruct((B,), jnp.float32),
        scratch_types=[
            pltpu.VMEM((b_per_w,), jnp.int32),
            pltpu.VMEM((b_per_w,), jnp.int32),
            pltpu.VMEM((O,), jnp.float32),
            pltpu.VMEM((C, D), jnp.float32),
            pltpu.VMEM((C, D), jnp.float32),
            pltpu.VMEM((C, D), jnp.float32),
            pltpu.VMEM((C, D), jnp.float32),
            pltpu.VMEM((b_per_w,), jnp.float32),
            pltpu.VMEM_SHARED((O, D), jnp.float32),
            pltpu.SemaphoreType.DMA,
            pltpu.SemaphoreType.DMA,
            pltpu.SemaphoreType.DMA,
            pltpu.SemaphoreType.DMA,
        ],
    )
    def mf_kernel(pid_hbm, oid_hbm, pemb_hbm, oemb_hbm, bias_hbm, out_hbm,
                  pid_v, oid_v, bias_v, prow0_v, prow1_v,
                  orow0_v, orow1_v,
                  out_v, otab_sh, sem_p0, sem_p1, sem_o0, sem_o1):
        wid = lax.axis_index("s") * NC + lax.axis_index("c")
        base = wid * b_per_w

        sid = lax.axis_index("s")

        c_pid = pltpu.async_copy(pid_hbm.at[pl.ds(base, b_per_w)], pid_v, sem_p0)
        c_oid = pltpu.async_copy(oid_hbm.at[pl.ds(base, b_per_w)], oid_v, sem_o0)
        c_bias = pltpu.async_copy(bias_hbm, bias_v, sem_p1)

        stage_rows = 128
        n_full = O // stage_rows
        tail_rows = O - n_full * stage_rows

        @pl.when(sid < n_full)
        def _stage_opening_table():
            pltpu.sync_copy(
                oemb_hbm.at[pl.ds(sid * stage_rows, stage_rows)],
                otab_sh.at[pl.ds(sid * stage_rows, stage_rows)])

        @pl.when(sid == n_full)
        def _stage_opening_table_tail():
            pltpu.sync_copy(
                oemb_hbm.at[pl.ds(n_full * stage_rows, tail_rows)],
                otab_sh.at[pl.ds(n_full * stage_rows, tail_rows)])

        c_pid.wait()
        c_oid.wait()
        c_bias.wait()

        lane = lax.iota(jnp.int32, L)
        lane0 = lane == 0

        prows = [prow0_v, prow1_v]
        orows = [orow0_v, orow1_v]
        sems_p = [sem_p0, sem_p1]
        sems_o = [sem_o0, sem_o1]
        NBUF = 2

        def issue_p(c):
            buf = c % NBUF
            return pltpu.async_copy(
                pemb_hbm.at[pid_v.at[pl.ds(c * C, C)]], prows[buf], sems_p[buf])

        def issue_o(c):
            buf = c % NBUF
            return pltpu.async_copy(
                otab_sh.at[oid_v.at[pl.ds(c * C, C)]], orows[buf], sems_o[buf])

        cp0 = issue_p(0)
        plsc.subcore_barrier()
        pending = [(cp0, issue_o(0))]
        for c in range(n_sub):
            buf = c % NBUF
            prow_v = prows[buf]
            orow_v = orows[buf]
            cp, co = pending.pop(0)
            cp.wait()
            co.wait()
            if c + NBUF - 1 < n_sub:
                nc = c + NBUF - 1
                pending.append((issue_p(nc), issue_o(nc)))

            @pl.loop(0, n_grp, unroll=1)
            def group_body(g):
                base_e = g * L
                goff = c * C + base_e
                ovec = oid_v[pl.ds(goff, L)]
                out_v[pl.ds(goff, L)] = plsc.load_gather(bias_v, [ovec])
                for u in range(L):
                    e = base_e + u
                    acc = prow_v[e, pl.ds(0, L)] * orow_v[e, pl.ds(0, L)]
                    for j in range(1, D // L):
                        acc += (prow_v[e, pl.ds(j * L, L)]
                                * orow_v[e, pl.ds(j * L, L)])
                    for m in (8, 4, 2, 1):
                        acc = acc + acc[lane ^ m]
                    plsc.addupdate_scatter(
                        out_v, [jnp.full((L,), goff + u, jnp.int32)], acc,
                        mask=lane0)

        pltpu.sync_copy(out_v, out_hbm.at[pl.ds(base, b_per_w)])

    return mf_kernel(
        player_ids.astype(jnp.int32),
        opening_ids.astype(jnp.int32),
        player_emb,
        opening_emb,
        opening_bias.reshape(O),
    )


# speedup vs baseline: 1.1937x; 1.1937x over previous
import functools

import jax
import jax.numpy as jnp
from jax import lax
from jax.experimental import pallas as pl
from jax.experimental.pallas import tpu as pltpu
from jax.experimental.pallas import tpu_sc as plsc


def kernel(player_ids, opening_ids, player_emb, opening_emb, opening_bias):
    B = player_ids.shape[0]
    D = player_emb.shape[1]
    O = opening_emb.shape[0]

    info = plsc.get_sparse_core_info()
    NC, NS, L = info.num_cores, info.num_subcores, info.num_lanes
    NW = NC * NS
    b_per_w = B // NW
    C = 128
    n_sub = b_per_w // C
    n_grp = C // L

    mesh = plsc.VectorSubcoreMesh(core_axis_name="c", subcore_axis_name="s")

    @functools.partial(
        pl.kernel,
        mesh=mesh,
        compiler_params=pltpu.CompilerParams(needs_layout_passes=False),
        out_type=jax.ShapeDtypeStruct((B,), jnp.float32),
        scratch_types=[
            pltpu.VMEM((b_per_w,), jnp.int32),
            pltpu.VMEM((b_per_w,), jnp.int32),
            pltpu.VMEM((O,), jnp.float32),
            pltpu.VMEM((C, D), jnp.float32),
            pltpu.VMEM((C, D), jnp.float32),
            pltpu.VMEM((C, D), jnp.float32),
            pltpu.VMEM((C, D), jnp.float32),
            pltpu.VMEM((b_per_w,), jnp.float32),
            pltpu.VMEM_SHARED((O, D), jnp.float32),
            pltpu.SemaphoreType.DMA,
            pltpu.SemaphoreType.DMA,
            pltpu.SemaphoreType.DMA,
            pltpu.SemaphoreType.DMA,
        ],
    )
    def mf_kernel(pid_hbm, oid_hbm, pemb_hbm, oemb_hbm, bias_hbm, out_hbm,
                  pid_v, oid_v, bias_v, prow0_v, prow1_v,
                  orow0_v, orow1_v,
                  out_v, otab_sh, sem_p0, sem_p1, sem_o0, sem_o1):
        wid = lax.axis_index("s") * NC + lax.axis_index("c")
        base = wid * b_per_w

        sid = lax.axis_index("s")

        c_pid = pltpu.async_copy(pid_hbm.at[pl.ds(base, b_per_w)], pid_v, sem_p0)
        c_oid = pltpu.async_copy(oid_hbm.at[pl.ds(base, b_per_w)], oid_v, sem_o0)
        c_bias = pltpu.async_copy(bias_hbm, bias_v, sem_p1)

        stage_rows = 128
        n_full = O // stage_rows
        tail_rows = O - n_full * stage_rows

        @pl.when(sid < n_full)
        def _stage_opening_table():
            pltpu.sync_copy(
                oemb_hbm.at[pl.ds(sid * stage_rows, stage_rows)],
                otab_sh.at[pl.ds(sid * stage_rows, stage_rows)])

        @pl.when(sid == n_full)
        def _stage_opening_table_tail():
            pltpu.sync_copy(
                oemb_hbm.at[pl.ds(n_full * stage_rows, tail_rows)],
                otab_sh.at[pl.ds(n_full * stage_rows, tail_rows)])

        c_pid.wait()
        c_oid.wait()
        c_bias.wait()

        lane = lax.iota(jnp.int32, L)
        quad_src = (lane & 3) * 4
        quad_id = lane >> 2
        quad_lead = (lane & 3) == 0
        lt4 = lane < 4
        lt8 = lane < 8
        lt12 = lane < 12

        prows = [prow0_v, prow1_v]
        orows = [orow0_v, orow1_v]
        sems_p = [sem_p0, sem_p1]
        sems_o = [sem_o0, sem_o1]
        NBUF = 2

        def issue_p(c):
            buf = c % NBUF
            return pltpu.async_copy(
                pemb_hbm.at[pid_v.at[pl.ds(c * C, C)]], prows[buf], sems_p[buf])

        def issue_o(c):
            buf = c % NBUF
            return pltpu.async_copy(
                otab_sh.at[oid_v.at[pl.ds(c * C, C)]], orows[buf], sems_o[buf])

        cp0 = issue_p(0)
        plsc.subcore_barrier()
        pending = [(cp0, issue_o(0))]
        for c in range(n_sub):
            buf = c % NBUF
            prow_v = prows[buf]
            orow_v = orows[buf]
            cp, co = pending.pop(0)
            cp.wait()
            co.wait()
            if c + NBUF - 1 < n_sub:
                nc = c + NBUF - 1
                pending.append((issue_p(nc), issue_o(nc)))

            @pl.loop(0, n_grp, unroll=1)
            def group_body(g):
                base_e = g * L
                goff = c * C + base_e
                ovec = oid_v[pl.ds(goff, L)]
                out_v[pl.ds(goff, L)] = plsc.load_gather(bias_v, [ovec])
                for t in range(L // 4):
                    accs = []
                    for u4 in range(4):
                        e = base_e + 4 * t + u4
                        acc = prow_v[e, pl.ds(0, L)] * orow_v[e, pl.ds(0, L)]
                        for j in range(1, D // L):
                            acc += (prow_v[e, pl.ds(j * L, L)]
                                    * orow_v[e, pl.ds(j * L, L)])
                        acc = acc + acc[lane ^ 1]
                        acc = acc + acc[lane ^ 2]
                        accs.append(acc)
                    pa, pb, pc, pd = (a[quad_src] for a in accs)
                    w = jnp.where(lt4, pa,
                                  jnp.where(lt8, pb, jnp.where(lt12, pc, pd)))
                    w = w + w[lane ^ 1]
                    w = w + w[lane ^ 2]
                    idx = jnp.full((L,), goff + 4 * t, jnp.int32) + quad_id
                    plsc.addupdate_scatter(out_v, [idx], w, mask=quad_lead)

        pltpu.sync_copy(out_v, out_hbm.at[pl.ds(base, b_per_w)])

    return mf_kernel(
        player_ids.astype(jnp.int32),
        opening_ids.astype(jnp.int32),
        player_emb,
        opening_emb,
        opening_bias.reshape(O),
    )
